# baseline (speedup 1.0000x reference)
import functools

import jax
import jax.numpy as jnp
from jax import lax
from jax.experimental import pallas as pl
from jax.experimental.pallas import tpu as pltpu
from jax.experimental.pallas import tpu_sc as plsc

SMALL = 1e-08

N_NODES = 10000
DIM = 128
N_TYPES = 4
EDGES_PER_TYPE = 80000

NC = 2
NS = 16
NW = NC * NS
N_EDGES = N_TYPES * EDGES_PER_TYPE
EPW = N_EDGES // NW
CHUNK = 96
ITERS = 105
EPWP = ITERS * CHUNK
TRASH = 10100
N_PAD = 10240
RPS = N_PAD // NS
CG = N_PAD // DIM
CG_PAD = 128

MM_R = 1000
CB_R = 1280


def _mm_body(x_ref, w_ref, b_ref, out_ref):
    y = lax.dot_general(
        x_ref[...], w_ref[0],
        dimension_numbers=(((1,), (1,)), ((), ())),
        preferred_element_type=jnp.float32,
    )
    out_ref[0] = y + b_ref[0]


def _build_table(node_states, w3, b2):
    return pl.pallas_call(
        _mm_body,
        grid=(N_TYPES, N_NODES // MM_R),
        in_specs=[
            pl.BlockSpec((MM_R, DIM), lambda i, nb: (nb, 0)),
            pl.BlockSpec((1, DIM, DIM), lambda i, nb: (i, 0, 0)),
            pl.BlockSpec((1, 1, DIM), lambda i, nb: (i, 0, 0)),
        ],
        out_specs=pl.BlockSpec((1, MM_R, DIM), lambda i, nb: (i, nb, 0)),
        out_shape=jax.ShapeDtypeStruct((N_TYPES, N_NODES, DIM), jnp.float32),
    )(node_states, w3, b2)


def _sc_body(table, gidx_hbm, dst_hbm, zeros_hbm, zerosv_hbm,
             out_msg, out_cnt,
             acc, idx_a, idx_b, idx_c, dst_a, dst_b, dst_c,
             rows_a, rows_b, rows_c, cnt_l,
             sem_ia, sem_ib, sem_ic, sem_a, sem_b, sem_c,
             sem_sa, sem_sb, sem_sc):
    c = lax.axis_index("c")
    s = lax.axis_index("s")
    wid = s * NC + c

    pltpu.sync_copy(zeros_hbm, acc.at[pl.ds(s * RPS, RPS)])
    pltpu.sync_copy(zerosv_hbm, cnt_l)
    plsc.subcore_barrier()

    ones16 = jnp.full((16,), 1.0, jnp.float32)
    base0 = pl.multiple_of(wid * EPWP, 8)

    def idx_load(k, idx_v, dst_v, sem):
        base = pl.multiple_of(base0 + k * CHUNK, 8)
        pltpu.make_async_copy(
            gidx_hbm.at[pl.ds(base, CHUNK)], idx_v, sem).start()
        pltpu.make_async_copy(
            dst_hbm.at[pl.ds(base, CHUNK)], dst_v, sem).start()
        pltpu.make_async_copy(
            gidx_hbm.at[pl.ds(base, CHUNK)], idx_v, sem).wait()
        pltpu.make_async_copy(
            dst_hbm.at[pl.ds(base, CHUNK)], dst_v, sem).wait()

    def gather(idx_v, rows, sem):
        return pltpu.make_async_copy(table.at[idx_v], rows, sem)

    def scatter_start(dst_v, rows, sem):
        pltpu.async_copy(rows, acc.at[dst_v], sem, add=True)

    def scatter_wait(dst_v, rows, sem):
        pltpu.make_async_copy(rows, acc.at[dst_v], sem).wait()

    def counts(dst_v):
        pass

    BUFS = ((idx_a, dst_a, rows_a, sem_ia, sem_a, sem_sa),
            (idx_b, dst_b, rows_b, sem_ib, sem_b, sem_sb),
            (idx_c, dst_c, rows_c, sem_ic, sem_c, sem_sc))

    def consume(k, buf):
        iv, dv, rv, si, sg, ss = buf
        gather(iv, rv, sg).wait()
        scatter_start(dv, rv, ss)
        counts(dv)

    def refill(k_next, buf):
        iv, dv, rv, si, sg, ss = buf
        scatter_wait(dv, rv, ss)
        idx_load(k_next, iv, dv, si)
        gather(iv, rv, sg).start()

    A, B, C = BUFS
    idx_load(0, A[0], A[1], A[3]); gather(A[0], A[2], A[4]).start()
    idx_load(1, B[0], B[1], B[3]); gather(B[0], B[2], B[4]).start()
    consume(0, A)
    idx_load(2, C[0], C[1], C[3]); gather(C[0], C[2], C[4]).start()
    consume(1, B)
    refill(3, A)
    consume(2, C)
    refill(4, B)

    @pl.loop(3, ITERS - 3, step=3)
    def _(k):
        consume(k, A)
        refill(k + 2, C)
        consume(k + 1, B)
        refill(k + 3, A)
        consume(k + 2, C)
        refill(k + 4, B)

    consume(ITERS - 3, A)
    refill(ITERS - 1, C)
    consume(ITERS - 2, B)
    consume(ITERS - 1, C)
    scatter_wait(A[1], A[2], A[5])
    scatter_wait(B[1], B[2], B[5])
    scatter_wait(C[1], C[2], C[5])
    plsc.subcore_barrier()

    pltpu.sync_copy(
        acc.at[pl.ds(s * RPS, RPS)],
        out_msg.at[pl.ds(c * N_PAD + s * RPS, RPS)],
    )
    pltpu.sync_copy(cnt_l, out_cnt.at[pl.ds(wid * N_PAD, N_PAD)])


_sc_scatter = functools.partial(
    pl.kernel,
    out_type=(
        jax.ShapeDtypeStruct((NC * N_PAD, DIM), jnp.float32),
        jax.ShapeDtypeStruct((NW * N_PAD,), jnp.float32),
    ),
    mesh=plsc.VectorSubcoreMesh(core_axis_name="c", subcore_axis_name="s"),
    compiler_params=pltpu.CompilerParams(needs_layout_passes=False),
    scratch_types=[
        pltpu.VMEM_SHARED((N_PAD, DIM), jnp.float32),
        pltpu.VMEM((CHUNK,), jnp.int32),
        pltpu.VMEM((CHUNK,), jnp.int32),
        pltpu.VMEM((CHUNK,), jnp.int32),
        pltpu.VMEM((CHUNK,), jnp.int32),
        pltpu.VMEM((CHUNK,), jnp.int32),
        pltpu.VMEM((CHUNK,), jnp.int32),
        pltpu.VMEM((CHUNK, DIM), jnp.float32),
        pltpu.VMEM((CHUNK, DIM), jnp.float32),
        pltpu.VMEM((CHUNK, DIM), jnp.float32),
        pltpu.VMEM((N_PAD,), jnp.float32),
        pltpu.SemaphoreType.DMA,
        pltpu.SemaphoreType.DMA,
        pltpu.SemaphoreType.DMA,
        pltpu.SemaphoreType.DMA,
        pltpu.SemaphoreType.DMA,
        pltpu.SemaphoreType.DMA,
        pltpu.SemaphoreType.DMA,
        pltpu.SemaphoreType.DMA,
        pltpu.SemaphoreType.DMA,
    ],
)(_sc_body)


def _comb_body(p_ref, c_ref, o_ref):
    nb = pl.program_id(0)
    ssum = p_ref[0] + p_ref[1]
    cnt = jnp.sum(
        c_ref[:, pl.ds(nb * (CB_R // DIM), CB_R // DIM), :], axis=0)
    div = jnp.where(cnt == 0.0, jnp.float32(1.0), cnt) + jnp.float32(SMALL)
    recip2d = 1.0 / div
    rr = jnp.broadcast_to(
        recip2d[:, None, :], (CB_R // DIM, DIM, DIM)).reshape(CB_R, DIM)
    i0 = lax.broadcasted_iota(jnp.int32, (CB_R, DIM), 0)
    i1 = lax.broadcasted_iota(jnp.int32, (CB_R, DIM), 1)
    m = lax.rem(i0, DIM) == i1
    recip = jnp.sum(jnp.where(m, rr, 0.0), axis=1, keepdims=True)
    o_ref[...] = ssum * recip


def _combine(partials, cnts):
    return pl.pallas_call(
        _comb_body,
        grid=(N_PAD // CB_R,),
        in_specs=[
            pl.BlockSpec((NC, CB_R, DIM), lambda nb: (0, nb, 0)),
            pl.BlockSpec((NW, CG, DIM), lambda nb: (0, 0, 0)),
        ],
        out_specs=pl.BlockSpec((CB_R, DIM), lambda nb: (nb, 0)),
        out_shape=jax.ShapeDtypeStruct((N_NODES, DIM), jnp.float32),
    )(partials, cnts)


def kernel(edge_lists, node_states, W, b):
    el = edge_lists.astype(jnp.int32)
    src = el[:, :, 0]
    dst = el[:, :, 1].reshape(-1)
    gidx = (src + jnp.arange(N_TYPES, dtype=jnp.int32)[:, None]
            * N_NODES).reshape(-1)

    w3 = W.reshape(N_TYPES, DIM, DIM)
    b2 = b.reshape(N_TYPES, 1, DIM)

    table = _build_table(node_states, w3, b2).reshape(N_TYPES * N_NODES, DIM)
    pad = EPWP - EPW
    gidx = jnp.concatenate(
        [gidx.reshape(NW, EPW),
         jnp.zeros((NW, pad), jnp.int32)], axis=1).reshape(-1)
    dst = jnp.concatenate(
        [dst.reshape(NW, EPW),
         jnp.full((NW, pad), TRASH, jnp.int32)], axis=1).reshape(-1)
    zeros = jnp.zeros((RPS, DIM), jnp.float32)
    zerosv = jnp.zeros((N_PAD,), jnp.float32)
    msgs, cnts = _sc_scatter(table, gidx, dst, zeros, zerosv)
    return _combine(msgs.reshape(NC, N_PAD, DIM), cnts.reshape(NW, CG, DIM))


# speedup vs baseline: 1.4223x; 1.4223x over previous
import functools

import jax
import jax.numpy as jnp
from jax import lax
from jax.experimental import pallas as pl
from jax.experimental.pallas import tpu as pltpu
from jax.experimental.pallas import tpu_sc as plsc

SMALL = 1e-08

N_NODES = 10000
DIM = 128
N_TYPES = 4
EDGES_PER_TYPE = 80000

NC = 2
NS = 16
NW = NC * NS
N_EDGES = N_TYPES * EDGES_PER_TYPE
EPW = N_EDGES // NW
CHUNK = 80
ITERS = 125
EPWP = ITERS * CHUNK
TRASH = 10100
N_PAD = 10240
RPS = N_PAD // NS
CG = N_PAD // DIM
CG_PAD = 128

MM_R = 1000
CB_R = 1280


def _mm_body(x_ref, w_ref, b_ref, out_ref):
    y = lax.dot_general(
        x_ref[...], w_ref[0],
        dimension_numbers=(((1,), (1,)), ((), ())),
        preferred_element_type=jnp.float32,
    )
    out_ref[0] = y + b_ref[0]


def _build_table(node_states, w3, b2):
    return pl.pallas_call(
        _mm_body,
        grid=(N_TYPES, N_NODES // MM_R),
        in_specs=[
            pl.BlockSpec((MM_R, DIM), lambda i, nb: (nb, 0)),
            pl.BlockSpec((1, DIM, DIM), lambda i, nb: (i, 0, 0)),
            pl.BlockSpec((1, 1, DIM), lambda i, nb: (i, 0, 0)),
        ],
        out_specs=pl.BlockSpec((1, MM_R, DIM), lambda i, nb: (i, nb, 0)),
        out_shape=jax.ShapeDtypeStruct((N_TYPES, N_NODES, DIM), jnp.float32),
    )(node_states, w3, b2)


def _sc_body(table, gidx_hbm, dst_hbm, zeros_hbm, zerosv_hbm,
             out_msg, out_cnt,
             acc, idx_a, idx_b, idx_c, dst_a, dst_b, dst_c,
             rows_a, rows_b, rows_c, cnt_l,
             sem_ia, sem_ib, sem_ic, sem_a, sem_b, sem_c,
             sem_sa, sem_sb, sem_sc):
    c = lax.axis_index("c")
    s = lax.axis_index("s")
    wid = s * NC + c

    ones16 = jnp.full((16,), 1.0, jnp.float32)
    base0 = pl.multiple_of(wid * EPWP, 8)

    def idx_load(k, idx_v, dst_v, sem):
        base = pl.multiple_of(base0 + k * CHUNK, 8)
        pltpu.make_async_copy(
            gidx_hbm.at[pl.ds(base, CHUNK)], idx_v, sem).start()
        pltpu.make_async_copy(
            dst_hbm.at[pl.ds(base, CHUNK)], dst_v, sem).start()
        pltpu.make_async_copy(
            gidx_hbm.at[pl.ds(base, CHUNK)], idx_v, sem).wait()
        pltpu.make_async_copy(
            dst_hbm.at[pl.ds(base, CHUNK)], dst_v, sem).wait()

    def gather(idx_v, rows, sem):
        return pltpu.make_async_copy(table.at[idx_v], rows, sem)

    def scatter_start(dst_v, rows, sem):
        pltpu.async_copy(rows, acc.at[dst_v], sem, add=True)

    def scatter_wait(dst_v, rows, sem):
        pltpu.make_async_copy(rows, acc.at[dst_v], sem).wait()

    def counts(dst_v):
        for g in range(CHUNK // 16):
            d16 = dst_v[pl.ds(g * 16, 16)]
            plsc.addupdate_scatter(cnt_l, [d16], ones16)

    BUFS = ((idx_a, dst_a, rows_a, sem_ia, sem_a, sem_sa),
            (idx_b, dst_b, rows_b, sem_ib, sem_b, sem_sb),
            (idx_c, dst_c, rows_c, sem_ic, sem_c, sem_sc))

    def consume(k, buf):
        iv, dv, rv, si, sg, ss = buf
        gather(iv, rv, sg).wait()
        scatter_start(dv, rv, ss)
        counts(dv)

    def refill(k_next, buf):
        iv, dv, rv, si, sg, ss = buf
        scatter_wait(dv, rv, ss)
        idx_load(k_next, iv, dv, si)
        gather(iv, rv, sg).start()

    A, B, C = BUFS
    idx_load(0, A[0], A[1], A[3]); gather(A[0], A[2], A[4]).start()
    idx_load(1, B[0], B[1], B[3]); gather(B[0], B[2], B[4]).start()
    pltpu.sync_copy(zeros_hbm, acc.at[pl.ds(s * RPS, RPS)])
    pltpu.sync_copy(zerosv_hbm, cnt_l)
    plsc.subcore_barrier()
    consume(0, A)
    idx_load(2, C[0], C[1], C[3]); gather(C[0], C[2], C[4]).start()
    consume(1, B)
    refill(3, A)
    consume(2, C)
    refill(4, B)

    @pl.loop(3, ITERS - 2, step=3)
    def _(k):
        consume(k, A)
        refill(k + 2, C)
        consume(k + 1, B)
        refill(k + 3, A)
        consume(k + 2, C)
        refill(k + 4, B)

    consume(ITERS - 2, A)
    consume(ITERS - 1, B)
    scatter_wait(A[1], A[2], A[5])
    scatter_wait(B[1], B[2], B[5])
    scatter_wait(C[1], C[2], C[5])
    plsc.subcore_barrier()

    pltpu.sync_copy(
        acc.at[pl.ds(s * RPS, RPS)],
        out_msg.at[pl.ds(c * N_PAD + s * RPS, RPS)],
    )
    pltpu.sync_copy(cnt_l, out_cnt.at[pl.ds(wid * N_PAD, N_PAD)])


_sc_scatter = functools.partial(
    pl.kernel,
    out_type=(
        jax.ShapeDtypeStruct((NC * N_PAD, DIM), jnp.float32),
        jax.ShapeDtypeStruct((NW * N_PAD,), jnp.float32),
    ),
    mesh=plsc.VectorSubcoreMesh(core_axis_name="c", subcore_axis_name="s"),
    compiler_params=pltpu.CompilerParams(needs_layout_passes=False),
    scratch_types=[
        pltpu.VMEM_SHARED((N_PAD, DIM), jnp.float32),
        pltpu.VMEM((CHUNK,), jnp.int32),
        pltpu.VMEM((CHUNK,), jnp.int32),
        pltpu.VMEM((CHUNK,), jnp.int32),
        pltpu.VMEM((CHUNK,), jnp.int32),
        pltpu.VMEM((CHUNK,), jnp.int32),
        pltpu.VMEM((CHUNK,), jnp.int32),
        pltpu.VMEM((CHUNK, DIM), jnp.float32),
        pltpu.VMEM((CHUNK, DIM), jnp.float32),
        pltpu.VMEM((CHUNK, DIM), jnp.float32),
        pltpu.VMEM((N_PAD,), jnp.float32),
        pltpu.SemaphoreType.DMA,
        pltpu.SemaphoreType.DMA,
        pltpu.SemaphoreType.DMA,
        pltpu.SemaphoreType.DMA,
        pltpu.SemaphoreType.DMA,
        pltpu.SemaphoreType.DMA,
        pltpu.SemaphoreType.DMA,
        pltpu.SemaphoreType.DMA,
        pltpu.SemaphoreType.DMA,
    ],
)(_sc_body)


def _comb_body(p_ref, c_ref, o_ref):
    nb = pl.program_id(0)
    ssum = p_ref[0] + p_ref[1]
    cnt = jnp.sum(
        c_ref[:, pl.ds(nb * (CB_R // DIM), CB_R // DIM), :], axis=0)
    div = jnp.where(cnt == 0.0, jnp.float32(1.0), cnt) + jnp.float32(SMALL)
    recip2d = 1.0 / div
    rr = jnp.broadcast_to(
        recip2d[:, None, :], (CB_R // DIM, DIM, DIM)).reshape(CB_R, DIM)
    i0 = lax.broadcasted_iota(jnp.int32, (CB_R, DIM), 0)
    i1 = lax.broadcasted_iota(jnp.int32, (CB_R, DIM), 1)
    m = lax.rem(i0, DIM) == i1
    recip = jnp.sum(jnp.where(m, rr, 0.0), axis=1, keepdims=True)
    o_ref[...] = ssum * recip


def _combine(partials, cnts):
    return pl.pallas_call(
        _comb_body,
        grid=(N_PAD // CB_R,),
        in_specs=[
            pl.BlockSpec((NC, CB_R, DIM), lambda nb: (0, nb, 0)),
            pl.BlockSpec((NW, CG, DIM), lambda nb: (0, 0, 0)),
        ],
        out_specs=pl.BlockSpec((CB_R, DIM), lambda nb: (nb, 0)),
        out_shape=jax.ShapeDtypeStruct((N_NODES, DIM), jnp.float32),
    )(partials, cnts)


def kernel(edge_lists, node_states, W, b):
    el = edge_lists.astype(jnp.int32)
    src = el[:, :, 0]
    dst = el[:, :, 1].reshape(-1)
    gidx = (src + jnp.arange(N_TYPES, dtype=jnp.int32)[:, None]
            * N_NODES).reshape(-1)

    w3 = W.reshape(N_TYPES, DIM, DIM)
    b2 = b.reshape(N_TYPES, 1, DIM)

    table = _build_table(node_states, w3, b2).reshape(N_TYPES * N_NODES, DIM)
    zeros = jnp.zeros((RPS, DIM), jnp.float32)
    zerosv = jnp.zeros((N_PAD,), jnp.float32)
    msgs, cnts = _sc_scatter(table, gidx, dst, zeros, zerosv)
    return _combine(msgs.reshape(NC, N_PAD, DIM), cnts.reshape(NW, CG, DIM))
